# baseline (speedup 1.0000x reference)
import jax
import jax.numpy as jnp
from jax.experimental import pallas as pl
from jax.experimental.pallas import tpu as pltpu

_NUM_CLASSES = 6
_KDIM = 128
_BC = 128
_K = 4


def _body(pr_ref, wq_ref, bq_ref, x_hbm, o_hbm, q_ref,
          inbuf, outbuf, insem, outsem):
    b_total, c_total = x_hbm.shape[0], x_hbm.shape[1]
    ncb = c_total // _BC
    nb = b_total * ncb

    def in_copy(t):
        b, c = divmod(t, ncb)
        return pltpu.make_async_copy(
            x_hbm.at[b, pl.ds(c * _BC, _BC)], inbuf.at[t % _K], insem.at[t % _K])

    def out_copy(t):
        b, c = divmod(t, ncb)
        return pltpu.make_async_copy(
            outbuf.at[t % _K], o_hbm.at[b, pl.ds(c * _BC, _BC)], outsem.at[t % _K])

    for t in range(_K):
        in_copy(t).start()

    for i in range(_NUM_CLASSES):
        for j in range(2):
            p = pr_ref[:, i, j, :]
            q = jax.lax.dot_general(
                p, wq_ref[i], (((1,), (1,)), ((), ())),
                preferred_element_type=jnp.float32,
            )
            q_ref[i * 2 + j] = q + bq_ref[i][None, :]

    for t in range(nb):
        s = t % _K
        in_copy(t).wait()
        if t >= _K:
            out_copy(t - _K).wait()
        outbuf[s] = inbuf[s] * jnp.float32(1.0 / 12.0)
        out_copy(t).start()
        if t + _K < nb:
            in_copy(t + _K).start()

    for t in range(nb - _K, nb):
        out_copy(t).wait()


def kernel(prototypes, assp_features, DomainTrain, Wk, bk, Wq, bq):
    b, c, h, w = assp_features.shape
    nc = prototypes.shape[1]
    pn = prototypes.shape[2]
    npairs = nc * pn

    out, q_all = pl.pallas_call(
        _body,
        in_specs=[
            pl.BlockSpec(memory_space=pltpu.VMEM),
            pl.BlockSpec(memory_space=pltpu.VMEM),
            pl.BlockSpec(memory_space=pltpu.VMEM),
            pl.BlockSpec(memory_space=pl.ANY),
        ],
        out_specs=[
            pl.BlockSpec(memory_space=pl.ANY),
            pl.BlockSpec(memory_space=pltpu.VMEM),
        ],
        out_shape=[
            jax.ShapeDtypeStruct((b, c, h, w), jnp.float32),
            jax.ShapeDtypeStruct((npairs, b, _KDIM), jnp.float32),
        ],
        scratch_shapes=[
            pltpu.VMEM((_K, _BC, h, w), jnp.float32),
            pltpu.VMEM((_K, _BC, h, w), jnp.float32),
            pltpu.SemaphoreType.DMA((_K,)),
            pltpu.SemaphoreType.DMA((_K,)),
        ],
    )(prototypes, Wq, bq, assp_features)

    return (out,) + tuple(q_all[p] for p in range(npairs))


# speedup vs baseline: 5.3946x; 5.3946x over previous
import jax
import jax.numpy as jnp
from jax.experimental import pallas as pl

_NUM_CLASSES = 6
_KDIM = 128
_BH = 16


def _fused_kernel(pr_ref, wq_ref, bq_ref, x_ref, o_ref, q_ref):
    o_ref[...] = x_ref[...] * jnp.float32(1.0 / 12.0)

    b = pl.program_id(0)
    hblk = pl.program_id(1)

    @pl.when(jnp.logical_and(b == 0, hblk == 0))
    def _():
        for i in range(_NUM_CLASSES):
            for j in range(2):
                p = pr_ref[:, i, j, :]
                q = jax.lax.dot_general(
                    p, wq_ref[i], (((1,), (1,)), ((), ())),
                    preferred_element_type=jnp.float32,
                )
                q_ref[i * 2 + j] = q + bq_ref[i][None, :]


def kernel(prototypes, assp_features, DomainTrain, Wk, bk, Wq, bq):
    b, c, h, w = assp_features.shape
    nc = prototypes.shape[1]
    pn = prototypes.shape[2]
    npairs = nc * pn

    xt = jnp.transpose(assp_features, (0, 2, 3, 1))
    grid = (b, h // _BH)
    out_t, q_all = pl.pallas_call(
        _fused_kernel,
        grid=grid,
        in_specs=[
            pl.BlockSpec(prototypes.shape, lambda bi, hi: (0, 0, 0, 0)),
            pl.BlockSpec(Wq.shape, lambda bi, hi: (0, 0, 0)),
            pl.BlockSpec(bq.shape, lambda bi, hi: (0, 0)),
            pl.BlockSpec((1, _BH, w, c), lambda bi, hi: (bi, hi, 0, 0)),
        ],
        out_specs=[
            pl.BlockSpec((1, _BH, w, c), lambda bi, hi: (bi, hi, 0, 0)),
            pl.BlockSpec((npairs, b, _KDIM), lambda bi, hi: (0, 0, 0)),
        ],
        out_shape=[
            jax.ShapeDtypeStruct((b, h, w, c), jnp.float32),
            jax.ShapeDtypeStruct((npairs, b, _KDIM), jnp.float32),
        ],
    )(prototypes, Wq, bq, xt)

    out = jnp.transpose(out_t, (0, 3, 1, 2))
    return (out,) + tuple(q_all[p] for p in range(npairs))


# speedup vs baseline: 6.3328x; 1.1739x over previous
import jax
import jax.numpy as jnp
from jax.experimental import pallas as pl

_NUM_CLASSES = 6
_KDIM = 128
_BH = 32


def _fused_kernel(pr_ref, wq_ref, bq_ref, x_ref, o_ref, q_ref):
    o_ref[...] = x_ref[...] * jnp.float32(1.0 / 12.0)

    b = pl.program_id(0)
    hblk = pl.program_id(1)

    @pl.when(jnp.logical_and(b == 0, hblk == 0))
    def _():
        for i in range(_NUM_CLASSES):
            for j in range(2):
                p = pr_ref[:, i, j, :]
                q = jax.lax.dot_general(
                    p, wq_ref[i], (((1,), (1,)), ((), ())),
                    preferred_element_type=jnp.float32,
                )
                q_ref[i * 2 + j] = q + bq_ref[i][None, :]


def kernel(prototypes, assp_features, DomainTrain, Wk, bk, Wq, bq):
    b, c, h, w = assp_features.shape
    nc = prototypes.shape[1]
    pn = prototypes.shape[2]
    npairs = nc * pn

    xt = jnp.transpose(assp_features, (0, 2, 3, 1))
    grid = (b, h // _BH)
    out_t, q_all = pl.pallas_call(
        _fused_kernel,
        grid=grid,
        in_specs=[
            pl.BlockSpec(prototypes.shape, lambda bi, hi: (0, 0, 0, 0)),
            pl.BlockSpec(Wq.shape, lambda bi, hi: (0, 0, 0)),
            pl.BlockSpec(bq.shape, lambda bi, hi: (0, 0)),
            pl.BlockSpec((1, _BH, w, c), lambda bi, hi: (bi, hi, 0, 0)),
        ],
        out_specs=[
            pl.BlockSpec((1, _BH, w, c), lambda bi, hi: (bi, hi, 0, 0)),
            pl.BlockSpec((npairs, b, _KDIM), lambda bi, hi: (0, 0, 0)),
        ],
        out_shape=[
            jax.ShapeDtypeStruct((b, h, w, c), jnp.float32),
            jax.ShapeDtypeStruct((npairs, b, _KDIM), jnp.float32),
        ],
    )(prototypes, Wq, bq, xt)

    out = jnp.transpose(out_t, (0, 3, 1, 2))
    return (out,) + tuple(q_all[p] for p in range(npairs))


# speedup vs baseline: 6.6182x; 1.0451x over previous
import jax
import jax.numpy as jnp
from jax.experimental import pallas as pl

_NUM_CLASSES = 6
_KDIM = 128
_BH = 64


def _fused_kernel(pr_ref, wq_ref, bq_ref, x_ref, o_ref, q_ref):
    o_ref[...] = x_ref[...] * jnp.float32(1.0 / 12.0)

    b = pl.program_id(0)
    hblk = pl.program_id(1)

    @pl.when(jnp.logical_and(b == 0, hblk == 0))
    def _():
        for i in range(_NUM_CLASSES):
            for j in range(2):
                p = pr_ref[:, i, j, :]
                q = jax.lax.dot_general(
                    p, wq_ref[i], (((1,), (1,)), ((), ())),
                    preferred_element_type=jnp.float32,
                )
                q_ref[i * 2 + j] = q + bq_ref[i][None, :]


def kernel(prototypes, assp_features, DomainTrain, Wk, bk, Wq, bq):
    b, c, h, w = assp_features.shape
    nc = prototypes.shape[1]
    pn = prototypes.shape[2]
    npairs = nc * pn

    xt = jnp.transpose(assp_features, (0, 2, 3, 1))
    grid = (b, h // _BH)
    out_t, q_all = pl.pallas_call(
        _fused_kernel,
        grid=grid,
        in_specs=[
            pl.BlockSpec(prototypes.shape, lambda bi, hi: (0, 0, 0, 0)),
            pl.BlockSpec(Wq.shape, lambda bi, hi: (0, 0, 0)),
            pl.BlockSpec(bq.shape, lambda bi, hi: (0, 0)),
            pl.BlockSpec((1, _BH, w, c), lambda bi, hi: (bi, hi, 0, 0)),
        ],
        out_specs=[
            pl.BlockSpec((1, _BH, w, c), lambda bi, hi: (bi, hi, 0, 0)),
            pl.BlockSpec((npairs, b, _KDIM), lambda bi, hi: (0, 0, 0)),
        ],
        out_shape=[
            jax.ShapeDtypeStruct((b, h, w, c), jnp.float32),
            jax.ShapeDtypeStruct((npairs, b, _KDIM), jnp.float32),
        ],
    )(prototypes, Wq, bq, xt)

    out = jnp.transpose(out_t, (0, 3, 1, 2))
    return (out,) + tuple(q_all[p] for p in range(npairs))


# speedup vs baseline: 7.0814x; 1.0700x over previous
import jax
import jax.numpy as jnp
from jax.experimental import pallas as pl

_NUM_CLASSES = 6
_KDIM = 128
_BH = 64
_BB = 2


def _fused_kernel(pr_ref, wq_ref, bq_ref, x_ref, o_ref, q_ref):
    o_ref[...] = x_ref[...] * jnp.float32(1.0 / 12.0)

    b = pl.program_id(0)
    hblk = pl.program_id(1)

    @pl.when(jnp.logical_and(b == 0, hblk == 0))
    def _():
        for i in range(_NUM_CLASSES):
            for j in range(2):
                p = pr_ref[:, i, j, :]
                q = jax.lax.dot_general(
                    p, wq_ref[i], (((1,), (1,)), ((), ())),
                    preferred_element_type=jnp.float32,
                )
                q_ref[i * 2 + j] = q + bq_ref[i][None, :]


def kernel(prototypes, assp_features, DomainTrain, Wk, bk, Wq, bq):
    b, c, h, w = assp_features.shape
    nc = prototypes.shape[1]
    pn = prototypes.shape[2]
    npairs = nc * pn

    xt = jnp.transpose(assp_features, (0, 2, 3, 1))
    grid = (b // _BB, h // _BH)
    out_t, q_all = pl.pallas_call(
        _fused_kernel,
        grid=grid,
        in_specs=[
            pl.BlockSpec(prototypes.shape, lambda bi, hi: (0, 0, 0, 0)),
            pl.BlockSpec(Wq.shape, lambda bi, hi: (0, 0, 0)),
            pl.BlockSpec(bq.shape, lambda bi, hi: (0, 0)),
            pl.BlockSpec((_BB, _BH, w, c), lambda bi, hi: (bi, hi, 0, 0)),
        ],
        out_specs=[
            pl.BlockSpec((_BB, _BH, w, c), lambda bi, hi: (bi, hi, 0, 0)),
            pl.BlockSpec((npairs, b, _KDIM), lambda bi, hi: (0, 0, 0)),
        ],
        out_shape=[
            jax.ShapeDtypeStruct((b, h, w, c), jnp.float32),
            jax.ShapeDtypeStruct((npairs, b, _KDIM), jnp.float32),
        ],
    )(prototypes, Wq, bq, xt)

    out = jnp.transpose(out_t, (0, 3, 1, 2))
    return (out,) + tuple(q_all[p] for p in range(npairs))
